# baseline (speedup 1.0000x reference)
import functools
import jax
import jax.numpy as jnp
from jax.experimental import pallas as pl
from jax.experimental.pallas import tpu as pltpu

N = 10000
E = 160000
DIN = 128
H = 256
DE = 16
G = 64
NB = 1000
EB = 640
N_NB = N // NB
N_EB = E // EB
F32 = jnp.float32


def _relu(v):
    return jnp.maximum(v, 0.0)


def _dot(a, b):
    return jnp.dot(a, b, preferred_element_type=F32)


def _dots(a, b):
    return jnp.dot(a, b, preferred_element_type=F32,
                   precision=jax.lax.Precision.HIGHEST)


def _k1_body(x_ref, w_ref, b_ref, o_ref):
    o_ref[...] = _dot(x_ref[...], w_ref[...]) + b_ref[...]


def _k1(x, w, b):
    return pl.pallas_call(
        _k1_body,
        grid=(N_NB,),
        in_specs=[
            pl.BlockSpec((NB, DIN), lambda i: (i, 0)),
            pl.BlockSpec((DIN, H), lambda i: (0, 0)),
            pl.BlockSpec((1, H), lambda i: (0, 0)),
        ],
        out_specs=pl.BlockSpec((NB, H), lambda i: (i, 0)),
        out_shape=jax.ShapeDtypeStruct((N, H), F32),
    )(x, w, b)


def _kedge_body(ea_ref, src_ref, dst_ref, h_ref, ew_ref, eb_ref, lw_ref,
                lb_ref, o_ref, elin_ref):
    pid = pl.program_id(0)

    @pl.when(pid == 0)
    def _():
        o_ref[...] = jnp.zeros((N, H), F32)

    e = _dot(ea_ref[...], ew_ref[...]) + eb_ref[...]
    elin_ref[...] = _dot(e, lw_ref[...]) + lb_ref[...]

    def body(j, _):
        s = src_ref[0, 0, j]
        d = dst_ref[0, 0, j]
        row = _relu(h_ref[pl.ds(s, 1), :] + elin_ref[pl.ds(j, 1), :])
        o_ref[pl.ds(d, 1), :] = o_ref[pl.ds(d, 1), :] + row
        return 0

    jax.lax.fori_loop(0, EB, body, 0)


def _kedge(ea, src3, dst3, h, ew, eb, lw, lb):
    return pl.pallas_call(
        _kedge_body,
        grid=(N_EB,),
        in_specs=[
            pl.BlockSpec((EB, DE), lambda i: (i, 0)),
            pl.BlockSpec((1, 1, EB), lambda i: (i, 0, 0),
                         memory_space=pltpu.SMEM),
            pl.BlockSpec((1, 1, EB), lambda i: (i, 0, 0),
                         memory_space=pltpu.SMEM),
            pl.BlockSpec((N, H), lambda i: (0, 0)),
            pl.BlockSpec((DE, H), lambda i: (0, 0)),
            pl.BlockSpec((1, H), lambda i: (0, 0)),
            pl.BlockSpec((H, H), lambda i: (0, 0)),
            pl.BlockSpec((1, H), lambda i: (0, 0)),
        ],
        out_specs=pl.BlockSpec((N, H), lambda i: (0, 0)),
        out_shape=jax.ShapeDtypeStruct((N, H), F32),
        scratch_shapes=[pltpu.VMEM((EB, H), F32)],
    )(ea, src3, dst3, h, ew, eb, lw, lb)


def _k3a_body(h_ref, a_ref, b3_ref, w1_ref, b1_ref, w2_ref, b2_ref,
              eps_ref, z_ref, m1_ref, cnt_out_ref, sz_ref, cnt_ref):
    pid = pl.program_id(0)

    @pl.when(pid == 0)
    def _():
        sz_ref[...] = jnp.zeros((G, H), F32)
        cnt_ref[...] = jnp.zeros((G, 128), F32)

    eps = eps_ref[0:1, 0:1]
    z = (1.0 + eps) * h_ref[...] + a_ref[...]
    z = _dot(_relu(_dot(z, w1_ref[...]) + b1_ref[...]), w2_ref[...]) + b2_ref[...]
    z_ref[...] = z

    bid = b3_ref[0, 0, :]
    onehot = (bid[:, None] ==
              jax.lax.broadcasted_iota(jnp.int32, (NB, G), 1)).astype(F32)
    oT = onehot.T
    sz_ref[...] = sz_ref[...] + _dots(oT, z)
    cnt_ref[...] = cnt_ref[...] + _dots(oT, jnp.ones((NB, 128), F32))

    @pl.when(pid == pl.num_programs(0) - 1)
    def _():
        cnt = cnt_ref[:, 0:1]
        m1_ref[...] = sz_ref[...] / cnt
        cnt_out_ref[...] = cnt_ref[...]


def _k3a(h, aggr, batch3, w1, b1, w2, b2, eps):
    return pl.pallas_call(
        _k3a_body,
        grid=(N_NB,),
        in_specs=[
            pl.BlockSpec((NB, H), lambda i: (i, 0)),
            pl.BlockSpec((NB, H), lambda i: (i, 0)),
            pl.BlockSpec((1, 1, NB), lambda i: (i, 0, 0)),
            pl.BlockSpec((H, H), lambda i: (0, 0)),
            pl.BlockSpec((1, H), lambda i: (0, 0)),
            pl.BlockSpec((H, H), lambda i: (0, 0)),
            pl.BlockSpec((1, H), lambda i: (0, 0)),
            pl.BlockSpec((1, 1), lambda i: (0, 0)),
        ],
        out_specs=[
            pl.BlockSpec((NB, H), lambda i: (i, 0)),
            pl.BlockSpec((G, H), lambda i: (0, 0)),
            pl.BlockSpec((G, 128), lambda i: (0, 0)),
        ],
        out_shape=[
            jax.ShapeDtypeStruct((N, H), F32),
            jax.ShapeDtypeStruct((G, H), F32),
            jax.ShapeDtypeStruct((G, 128), F32),
        ],
        scratch_shapes=[pltpu.VMEM((G, H), F32), pltpu.VMEM((G, 128), F32)],
    )(h, aggr, batch3, w1, b1, w2, b2, eps)


def _k3v_body(z_ref, b3_ref, m1_ref, cnt_ref, ga_ref, var_ref, acc_ref):
    pid = pl.program_id(0)

    @pl.when(pid == 0)
    def _():
        acc_ref[...] = jnp.zeros((G, H), F32)

    bid = b3_ref[0, 0, :]
    onehot = (bid[:, None] ==
              jax.lax.broadcasted_iota(jnp.int32, (NB, G), 1)).astype(F32)
    m1g = _dots(onehot, m1_ref[...])
    zc = z_ref[...] - ga_ref[...] * m1g
    acc_ref[...] = acc_ref[...] + _dots(onehot.T, zc * zc)

    @pl.when(pid == pl.num_programs(0) - 1)
    def _():
        var_ref[...] = acc_ref[...] / cnt_ref[:, 0:1]


def _k3v(z, batch3, m1, cnt, ga):
    return pl.pallas_call(
        _k3v_body,
        grid=(N_NB,),
        in_specs=[
            pl.BlockSpec((NB, H), lambda i: (i, 0)),
            pl.BlockSpec((1, 1, NB), lambda i: (i, 0, 0)),
            pl.BlockSpec((G, H), lambda i: (0, 0)),
            pl.BlockSpec((G, 128), lambda i: (0, 0)),
            pl.BlockSpec((1, H), lambda i: (0, 0)),
        ],
        out_specs=pl.BlockSpec((G, H), lambda i: (0, 0)),
        out_shape=jax.ShapeDtypeStruct((G, H), F32),
        scratch_shapes=[pltpu.VMEM((G, H), F32)],
    )(z, batch3, m1, cnt, ga)


def _k3b_body(z_ref, b3_ref, m1_ref, var_ref, ga_ref, gw_ref, gb_ref, o_ref):
    bid = b3_ref[0, 0, :]
    onehot = (bid[:, None] ==
              jax.lax.broadcasted_iota(jnp.int32, (NB, G), 1)).astype(F32)
    m1g = _dots(onehot, m1_ref[...])
    varg = _dots(onehot, var_ref[...])
    zc = z_ref[...] - ga_ref[...] * m1g
    o_ref[...] = _relu(gw_ref[...] * zc / jnp.sqrt(varg + 1e-5)
                       + gb_ref[...])


def _k3b(z, batch3, m1, var, ga, gw, gb):
    return pl.pallas_call(
        _k3b_body,
        grid=(N_NB,),
        in_specs=[
            pl.BlockSpec((NB, H), lambda i: (i, 0)),
            pl.BlockSpec((1, 1, NB), lambda i: (i, 0, 0)),
            pl.BlockSpec((G, H), lambda i: (0, 0)),
            pl.BlockSpec((G, H), lambda i: (0, 0)),
            pl.BlockSpec((1, H), lambda i: (0, 0)),
            pl.BlockSpec((1, H), lambda i: (0, 0)),
            pl.BlockSpec((1, H), lambda i: (0, 0)),
        ],
        out_specs=pl.BlockSpec((NB, H), lambda i: (i, 0)),
        out_shape=jax.ShapeDtypeStruct((N, H), F32),
    )(z, batch3, m1, var, ga, gw, gb)


def _kstats_body(x8_ref, o_ref):
    x8 = x8_ref[...]
    mx = jnp.max(x8, axis=0, keepdims=True)
    sm = jnp.sum(x8, axis=0, keepdims=True)
    s2 = jnp.sum(x8 * x8, axis=0, keepdims=True)
    o_ref[0:1, :] = mx
    o_ref[1:2, :] = sm
    o_ref[2:3, :] = s2
    o_ref[3:8, :] = jnp.zeros((5, 8), F32)


def _kstats(x8):
    return pl.pallas_call(
        _kstats_body,
        in_specs=[pl.BlockSpec((N, 8), lambda: (0, 0))],
        out_specs=pl.BlockSpec((8, 8), lambda: (0, 0)),
        out_shape=jax.ShapeDtypeStruct((8, 8), F32),
    )(x8)


def _khead_body(h_ref, x8_ref, b3_ref, st_ref, aw1h_ref, aw1x_ref, ab1_ref,
                aw2_ref, ab2_ref, iw1_ref, ib1_ref, iw2_ref, ib2_ref,
                n1w1_ref, n1b1_ref, n1w2_ref, n1b2_ref,
                n2w1_ref, n2b1_ref, n2w2_ref, n2b2_ref,
                hub_ref, patt_ref, attn_ref, l1_ref, l2_ref, ge_ref):
    pid = pl.program_id(0)

    @pl.when(pid == 0)
    def _():
        ge_ref[...] = jnp.zeros((G, H), F32)

    x8 = x8_ref[...]
    fi = x8[:, 0:1]; fo = x8[:, 1:2]; pr = x8[:, 2:3]; loc = x8[:, 3:4]
    nc = x8[:, 4:5]; inst = x8[:, 5:6]; ab = x8[:, 6:7]
    eps = 1e-8
    hub = (0.25 * fi / (st_ref[0:1, 0:1] + eps)
           + 0.25 * fo / (st_ref[0:1, 1:2] + eps)
           + 0.2 * pr / (st_ref[0:1, 2:3] + eps)
           + 0.15 * loc / (st_ref[0:1, 3:4] + eps)
           + 0.1 * nc / (st_ref[0:1, 4:5] + eps)
           + 0.05 * inst)
    hub = jnp.clip(hub, 0.0, 1.0)
    hub_ref[...] = hub

    h = h_ref[...]
    a1 = _relu(_dot(h, aw1h_ref[...]) + hub * aw1x_ref[...] + ab1_ref[...])
    attn = jax.nn.sigmoid(_dot(a1, aw2_ref[...]) + ab2_ref[...])
    attn_ref[...] = attn
    att = h * (1.0 + attn)

    fi_th = st_ref[1:2, 0:1]; fo_th = st_ref[1:2, 1:2]
    loc_th = st_ref[1:2, 2:3]; fo_med = st_ref[1:2, 3:4]
    f1 = jnp.float32(1.0); f0 = jnp.float32(0.0)
    p0 = jnp.where((fo > fo_th) & (inst > 0.7) & (hub > 0.6), f1, f0)
    p1 = jnp.where((loc > loc_th) & (hub > 0.5), f1, f0)
    p2 = jnp.where((fi > fi_th) & (fo > fo_th) & (hub > 0.7), f1, f0)
    p3 = jnp.where((fo > fo_th) & (ab < 0.3) & (hub > 0.6), f1, f0)
    p4 = jnp.where((fo > fo_med) & (inst > 0.5) & (hub > 0.4), f1, f0)
    p5 = jnp.where((fo > 0) | (fi > 0), f1, f0)
    patt_ref[...] = jnp.concatenate([p0, p1, p2, p3, p4, p5], axis=1)

    imp = jax.nn.sigmoid(
        _dot(_relu(_dot(att, iw1_ref[...]) + ib1_ref[...]), iw2_ref[...])
        + ib2_ref[...])
    comb = imp * (1.0 + 2.0 * hub)

    bid = b3_ref[0, 0, :]
    onehot = (bid[:, None] ==
              jax.lax.broadcasted_iota(jnp.int32, (NB, G), 1)).astype(F32)
    ge_ref[...] = ge_ref[...] + _dots(onehot.T, att * comb)

    l1_ref[...] = (_dot(_relu(_dot(att, n1w1_ref[...]) + n1b1_ref[...]),
                        n1w2_ref[...]) + n1b2_ref[...])
    l2_ref[...] = (_dot(_relu(_dot(att, n2w1_ref[...]) + n2b1_ref[...]),
                        n2w2_ref[...]) + n2b2_ref[...])


def _khead(h, x8, batch3, st, p):
    M = H // 2
    full = lambda shp: pl.BlockSpec(shp, lambda i: tuple(0 for _ in shp))
    return pl.pallas_call(
        _khead_body,
        grid=(N_NB,),
        in_specs=[
            pl.BlockSpec((NB, H), lambda i: (i, 0)),
            pl.BlockSpec((NB, 8), lambda i: (i, 0)),
            pl.BlockSpec((1, 1, NB), lambda i: (i, 0, 0)),
            full((8, 8)),
            full((H, M)), full((1, M)), full((1, M)),
            full((M, 1)), full((1, 1)),
            full((H, M)), full((1, M)), full((M, 1)), full((1, 1)),
            full((H, M)), full((1, M)), full((M, 1)), full((1, 1)),
            full((H, M)), full((1, M)), full((M, 1)), full((1, 1)),
        ],
        out_specs=[
            pl.BlockSpec((NB, 1), lambda i: (i, 0)),
            pl.BlockSpec((NB, 6), lambda i: (i, 0)),
            pl.BlockSpec((NB, 1), lambda i: (i, 0)),
            pl.BlockSpec((NB, 1), lambda i: (i, 0)),
            pl.BlockSpec((NB, 1), lambda i: (i, 0)),
            pl.BlockSpec((G, H), lambda i: (0, 0)),
        ],
        out_shape=[
            jax.ShapeDtypeStruct((N, 1), F32),
            jax.ShapeDtypeStruct((N, 6), F32),
            jax.ShapeDtypeStruct((N, 1), F32),
            jax.ShapeDtypeStruct((N, 1), F32),
            jax.ShapeDtypeStruct((N, 1), F32),
            jax.ShapeDtypeStruct((G, H), F32),
        ],
    )(h, x8, batch3, st,
      p['att_w1'][:H], p['att_w1'][H:H + 1], p['att_b1'][None],
      p['att_w2'], p['att_b2'][None],
      p['imp_w1'], p['imp_b1'][None], p['imp_w2'], p['imp_b2'][None],
      p['n1_w1'], p['n1_b1'][None], p['n1_w2'], p['n1_b2'][None],
      p['n2_w1'], p['n2_b1'][None], p['n2_w2'], p['n2_b2'][None])


def _khead3_body(l1_ref, l2_ref, ge_ref, pw1_ref, pb1_ref, pw2_ref, pb2_ref,
                 tw1_ref, tb1_ref, tw2_ref, tb2_ref,
                 pi1_ref, pi2_ref, l3_ref, l4_ref, pi3_ref, pi4_ref):
    def soft_col(v):
        m = jnp.max(v, axis=0, keepdims=True)
        e = jnp.exp(v - m)
        return e / jnp.sum(e, axis=0, keepdims=True)

    pi1_ref[...] = soft_col(l1_ref[...])
    pi2_ref[...] = soft_col(l2_ref[...])

    ge = ge_ref[...]
    l3 = (_dot(_relu(_dot(ge, pw1_ref[...]) + pb1_ref[...]), pw2_ref[...])
          + pb2_ref[...])
    l4 = (_dot(_relu(_dot(ge, tw1_ref[...]) + tb1_ref[...]), tw2_ref[...])
          + tb2_ref[...])
    l3_ref[...] = l3
    l4_ref[...] = l4

    def soft_row(v):
        m = jnp.max(v, axis=1, keepdims=True)
        e = jnp.exp(v - m)
        return e / jnp.sum(e, axis=1, keepdims=True)

    pi3_ref[...] = soft_row(l3)
    pi4_ref[...] = soft_row(l4)


def _khead3(l1, l2, ge, p):
    M = H // 2
    full = lambda shp: pl.BlockSpec(shp, lambda: tuple(0 for _ in shp))
    return pl.pallas_call(
        _khead3_body,
        in_specs=[
            full((N, 1)), full((N, 1)), full((G, H)),
            full((H, M)), full((1, M)), full((M, 6)), full((1, 6)),
            full((H, M)), full((1, M)), full((M, 2)), full((1, 2)),
        ],
        out_specs=[
            full((N, 1)), full((N, 1)), full((G, 6)), full((G, 2)),
            full((G, 6)), full((G, 2)),
        ],
        out_shape=[
            jax.ShapeDtypeStruct((N, 1), F32),
            jax.ShapeDtypeStruct((N, 1), F32),
            jax.ShapeDtypeStruct((G, 6), F32),
            jax.ShapeDtypeStruct((G, 2), F32),
            jax.ShapeDtypeStruct((G, 6), F32),
            jax.ShapeDtypeStruct((G, 2), F32),
        ],
    )(l1, l2, ge,
      p['pat_w1'], p['pat_b1'][None], p['pat_w2'], p['pat_b2'][None],
      p['term_w1'], p['term_b1'][None], p['term_w2'], p['term_b2'][None])


def kernel(x, edge_attr, params, edge_index, batch):
    p = params
    src3 = edge_index[0].reshape(N_EB, 1, EB)
    dst3 = edge_index[1].reshape(N_EB, 1, EB)
    batch3 = batch.reshape(N_NB, 1, NB)
    x8 = x[:, :8]

    h = _k1(x, p['node_w'], p['node_b'][None])

    for i in range(L_LAYERS):
        aggr = _kedge(edge_attr, src3, dst3, h,
                      p['edge_w'], p['edge_b'][None],
                      p['lin_e_w%d' % i], p['lin_e_b%d' % i][None])
        z, m1, cnt = _k3a(h, aggr, batch3,
                          p['mlp_w1_%d' % i], p['mlp_b1_%d' % i][None],
                          p['mlp_w2_%d' % i], p['mlp_b2_%d' % i][None],
                          p['gine_eps%d' % i].reshape(1, 1))
        var = _k3v(z, batch3, m1, cnt, p['gn_a%d' % i][None])
        h = _k3b(z, batch3, m1, var, p['gn_a%d' % i][None],
                 p['gn_w%d' % i][None], p['gn_b%d' % i][None])

    fi = x[:, 0]; fo = x[:, 1]; pr = x[:, 2]; loc = x[:, 3]; nc = x[:, 4]
    fo_med = jnp.median(fo)
    fo_th = fo_med + jnp.std(fo)
    fi_th = jnp.median(fi) + jnp.std(fi)
    loc_th = jnp.quantile(loc, 0.75)
    st = jnp.zeros((8, 8), F32)
    st = st.at[0, 0].set(fi.max()).at[0, 1].set(fo.max())
    st = st.at[0, 2].set(pr.max()).at[0, 3].set(loc.max())
    st = st.at[0, 4].set(nc.max())
    st = st.at[1, 0].set(fi_th).at[1, 1].set(fo_th)
    st = st.at[1, 2].set(loc_th).at[1, 3].set(fo_med)

    hub, patt, attn, l1, l2, ge = _khead(h, x8, batch3, st, p)
    pi1, pi2, l3, l4, pi3, pi4 = _khead3(l1, l2, ge, p)

    return (pi1[:, 0], pi2[:, 0], pi3, pi4, l1[:, 0], l2[:, 0], l3, l4,
            hub[:, 0], patt, attn[:, 0])


L_LAYERS = 3
